# baseline (speedup 1.0000x reference)
import functools

import jax
import jax.numpy as jnp
from jax import lax
from jax.experimental import pallas as pl
from jax.experimental.pallas import tpu as pltpu
from jax.experimental.pallas import tpu_sc as plsc

N = 10000
E = 160000
D = 256

NC = 2
NS = 16
NT = NC * NS
LANES = 16

ROWS_T = 320
TRASH = ROWS_T
NP = NT * ROWS_T
W = 1600
NW = E // W
EC = 32
PIPE = 2
CAP = W + 4 * LANES
LAST_ROWS = N - (NT - 1) * ROWS_T
QPAD = 384


PSH = 16384


def _tc_body(obs_ref, wfc_ref, bfc_ref, w12_ref, ba_ref, ei_ref,
             feat_ref, pq_ref, pk_ref):
    f = lax.dot_general(
        obs_ref[...], wfc_ref[...], (((1,), (1,)), ((), ())),
        preferred_element_type=jnp.float32, precision=lax.Precision.HIGHEST)
    f = f + bfc_ref[...]
    feat_ref[...] = f
    pq = lax.dot_general(
        w12_ref[...], f, (((1,), (1,)), ((), ())),
        preferred_element_type=jnp.float32, precision=lax.Precision.HIGHEST)
    b = ba_ref[0, 0]
    rowmask = lax.broadcasted_iota(jnp.int32, (8, N), 0) == 0
    pq = pq + jnp.where(rowmask, b, 0.0)
    pq_ref[...] = jnp.concatenate(
        [pq, jnp.zeros((8, NP - N), jnp.float32)], axis=1)
    ei = ei_ref[...]
    pk_ref[...] = ei[0:1, :] * PSH + ei[1:2, :]


def _tc_features(obs, wfc, bfc, w12, ba, ei):
    return pl.pallas_call(
        _tc_body,
        in_specs=[
            pl.BlockSpec((N, D), lambda: (0, 0)),
            pl.BlockSpec((D, D), lambda: (0, 0)),
            pl.BlockSpec((1, D), lambda: (0, 0)),
            pl.BlockSpec((8, D), lambda: (0, 0)),
            pl.BlockSpec(memory_space=pltpu.SMEM),
            pl.BlockSpec((2, E), lambda: (0, 0)),
        ],
        out_specs=[
            pl.BlockSpec((N, D), lambda: (0, 0)),
            pl.BlockSpec((8, NP), lambda: (0, 0)),
            pl.BlockSpec((1, E), lambda: (0, 0)),
        ],
        out_shape=[
            jax.ShapeDtypeStruct((N, D), jnp.float32),
            jax.ShapeDtypeStruct((8, NP), jnp.float32),
            jax.ShapeDtypeStruct((1, E), jnp.int32),
        ],
    )(obs, wfc, bfc, w12, ba, ei)


SCAN_UNROLL = 5


@functools.partial(
    pl.kernel,
    out_type=jax.ShapeDtypeStruct((N, D), jnp.float32),
    mesh=plsc.VectorSubcoreMesh(core_axis_name="c", subcore_axis_name="s"),
    compiler_params=pltpu.CompilerParams(needs_layout_passes=False),
    scratch_types=[
        pltpu.VMEM((ROWS_T + 1, D), jnp.float32),
        pltpu.VMEM((NP,), jnp.float32),
        pltpu.VMEM((QPAD,), jnp.float32),
        pltpu.VMEM((2 * W,), jnp.int32),
        pltpu.VMEM((CAP,), jnp.int32),
        pltpu.VMEM((PIPE * EC, D), jnp.float32),
        pltpu.VMEM((PIPE * EC,), jnp.float32),
        pltpu.VMEM((PIPE * EC,), jnp.int32),
        pltpu.VMEM((PIPE * EC,), jnp.int32),
        pltpu.SemaphoreType.DMA((2,)),
        pltpu.SemaphoreType.DMA((PIPE,)),
    ],
)
def _sc_gat(feat_hbm, pq_hbm, qt_hbm, pk_hbm, out_hbm,
            acc, p_v, q_v, pk_w, esel, rows_v, alpha_v, dloc_v,
            gidx_v, wsem, gsem):
    c = lax.axis_index("c")
    s = lax.axis_index("s")
    wid = c * NS + s
    lo = wid * ROWS_T
    hi = lo + ROWS_T
    iota = lax.broadcasted_iota(jnp.int32, (LANES,), 0)
    zerosf = jnp.zeros((LANES,), jnp.float32)
    zeros16 = jnp.zeros((LANES,), jnp.int32)
    cols = [jj * LANES + iota for jj in range(D // LANES)]

    def zrow(r, carry):
        for j in range(D // LANES):
            acc[r, pl.ds(j * LANES, LANES)] = zerosf
        return carry
    lax.fori_loop(0, ROWS_T + 1, zrow, 0)

    pltpu.sync_copy(pq_hbm.at[0], p_v)
    pltpu.sync_copy(qt_hbm.at[wid], q_v)

    def start_gather(par):
        pltpu.async_copy(feat_hbm.at[gidx_v.at[pl.ds(par * EC, EC)]],
                         rows_v.at[pl.ds(par * EC, EC)], gsem.at[par])

    def accumulate(par):
        pltpu.make_async_copy(feat_hbm.at[pl.ds(0, EC)],
                              rows_v.at[pl.ds(par * EC, EC)],
                              gsem.at[par]).wait()

        def srow(e4, carry):
            pes = [par * EC + e4 * 4 + u for u in range(4)]
            abs_ = [plsc.load_gather(alpha_v, [zeros16 + pe])
                    for pe in pes]
            dlbs = [plsc.load_gather(dloc_v, [zeros16 + pe])
                    for pe in pes]
            vs = [[rows_v[pe, pl.ds(jj * LANES, LANES)] * abs_[u]
                   for jj in range(D // LANES)]
                  for u, pe in enumerate(pes)]
            for u in range(4):
                for jj in range(D // LANES):
                    plsc.addupdate_scatter(acc, [dlbs[u], cols[jj]],
                                           vs[u][jj])
            return carry
        lax.fori_loop(0, EC // 4, srow, 0)

    def prep_groups(par, base, npend):
        for g in range(EC // LANES):
            pk16 = esel[pl.ds(base + g * LANES, LANES)]
            s16 = lax.shift_right_logical(pk16, 14)
            dl16 = (pk16 & (PSH - 1)) - lo
            if npend is not None:
                valid = (g * LANES + iota) < npend
                s16 = jnp.where(valid, s16, 0)
                dl16 = jnp.where(valid, dl16, TRASH)
            pg = plsc.load_gather(p_v, [s16])
            qg = plsc.load_gather(q_v, [dl16])
            a = 1.0 / (1.0 + jnp.exp(-(pg + qg)))
            if npend is not None:
                a = jnp.where(valid, a, 0.0)
            alpha_v[pl.ds(par * EC + g * LANES, LANES)] = a
            dloc_v[pl.ds(par * EC + g * LANES, LANES)] = dl16
            gidx_v[pl.ds(par * EC + g * LANES, LANES)] = s16

    def prep(par, base):
        prep_groups(par, base, None)

    def prep_masked(par, npend):
        prep_groups(par, jnp.int32(0), npend)

    pltpu.async_copy(pk_hbm.at[pl.ds(0, W)], pk_w.at[pl.ds(0, W)],
                     wsem.at[0])

    def window(w, st):
        pend0, nq0, head0 = st
        cur = w % 2
        woff = w * W
        pltpu.make_async_copy(pk_hbm.at[pl.ds(woff, W)],
                              pk_w.at[pl.ds(cur * W, W)],
                              wsem.at[cur]).wait()

        @pl.when(w + 1 < NW)
        def _prefetch():
            noff = (w + 1) * W
            pltpu.async_copy(pk_hbm.at[pl.ds(noff, W)],
                             pk_w.at[pl.ds((1 - cur) * W, W)],
                             wsem.at[1 - cur])

        def scan(g, pendc):
            pks, ms, offs = [], [], []
            for u in range(SCAN_UNROLL):
                pk16 = pk_w[pl.ds(cur * W + (g * SCAN_UNROLL + u) * LANES,
                                  LANES)]
                d16 = pk16 & (PSH - 1)
                m = (d16 >= lo) & (d16 < hi)
                pks.append(pk16)
                ms.append(m)
                offs.append(pendc)
                pendc = pendc + plsc.all_reduce_population_count(m)[0]
            for u in range(SCAN_UNROLL):
                plsc.store_compressed(esel.at[pl.ds(offs[u], LANES)],
                                      pks[u], mask=ms[u])
            return pendc
        pend = lax.fori_loop(0, W // (LANES * SCAN_UNROLL), scan, pend0)

        def pcond(pst):
            return pst[0] >= EC

        def pbody(pst):
            pendc, basec, nqc, headc = pst

            @pl.when(nqc == PIPE)
            def _drain():
                accumulate(headc)
            nq2 = jnp.where(nqc == PIPE, nqc - 1, nqc)
            prep(headc, basec)
            start_gather(headc)
            return (pendc - EC, basec + EC, nq2 + 1,
                    (headc + 1) & (PIPE - 1))
        pend, base, nq, head = lax.while_loop(
            pcond, pbody, (pend, jnp.int32(0), nq0, head0))

        for k in range(EC // LANES):
            idx16 = base + k * LANES + iota
            ev = plsc.load_gather(esel, [idx16])
            esel[pl.ds(k * LANES, LANES)] = ev
        return (pend, nq, head)

    pend, nq, head = lax.fori_loop(
        0, NW, window, (jnp.int32(0), jnp.int32(0), jnp.int32(0)))

    @pl.when(pend > 0)
    def _tail():
        @pl.when(nq == PIPE)
        def _oldest():
            accumulate(head)
        prep_masked(head, pend)
        start_gather(head)

    nq = jnp.where(pend > 0, jnp.where(nq == PIPE, nq, nq + 1), nq)
    head = jnp.where(pend > 0, (head + 1) & (PIPE - 1), head)

    def dbody(k, carry):
        accumulate((head + PIPE - nq + k) & (PIPE - 1))
        return carry
    lax.fori_loop(0, nq, dbody, 0)

    @pl.when(wid < NT - 1)
    def _full():
        pltpu.sync_copy(acc.at[pl.ds(0, ROWS_T)],
                        out_hbm.at[pl.ds(lo, ROWS_T)])

    @pl.when(wid == NT - 1)
    def _last():
        pltpu.sync_copy(acc.at[pl.ds(0, LAST_ROWS)],
                        out_hbm.at[pl.ds(lo, LAST_ROWS)])


def kernel(observations, edge_index, W_fc, b_fc, W_attn, b_attn):
    w1 = W_attn[0, :D]
    w2 = W_attn[0, D:]
    w12 = jnp.zeros((8, D), jnp.float32).at[0].set(w1).at[1].set(w2)
    feat, pq, pk = _tc_features(
        observations, W_fc, b_fc.reshape(1, D), w12, b_attn.reshape(1, 1),
        edge_index)
    qt = jnp.pad(pq[1].reshape(NT, ROWS_T),
                 ((0, 0), (0, QPAD - ROWS_T)))
    return _sc_gat(feat, pq, qt, pk.reshape(E))


# speedup vs baseline: 1.1972x; 1.1972x over previous
import functools

import jax
import jax.numpy as jnp
from jax import lax
from jax.experimental import pallas as pl
from jax.experimental.pallas import tpu as pltpu
from jax.experimental.pallas import tpu_sc as plsc

N = 10000
E = 160000
D = 256

NC = 2
NS = 16
NT = NC * NS
LANES = 16

ROWS_T = 320
TRASH = ROWS_T
NP = NT * ROWS_T
W = 1600
NW = E // W
EC = 32
PIPE = 2
CAP = W + 4 * LANES
LAST_ROWS = N - (NT - 1) * ROWS_T
QPAD = 384


PSH = 16384


def _tc_body(obs_ref, wfc_ref, bfc_ref, w12_ref, ba_ref, ei_ref,
             feat_ref, pq_ref, pk_ref):
    f = lax.dot_general(
        obs_ref[...], wfc_ref[...], (((1,), (1,)), ((), ())),
        preferred_element_type=jnp.float32, precision=lax.Precision.HIGHEST)
    f = f + bfc_ref[...]
    feat_ref[...] = f
    pq = lax.dot_general(
        w12_ref[...], f, (((1,), (1,)), ((), ())),
        preferred_element_type=jnp.float32, precision=lax.Precision.HIGHEST)
    b = ba_ref[0, 0]
    rowmask = lax.broadcasted_iota(jnp.int32, (8, N), 0) == 0
    pq = pq + jnp.where(rowmask, b, 0.0)
    pq_ref[...] = jnp.concatenate(
        [pq, jnp.zeros((8, NP - N), jnp.float32)], axis=1)
    ei = ei_ref[...]
    pk_ref[...] = ei[0:1, :] * PSH + ei[1:2, :]


def _tc_features(obs, wfc, bfc, w12, ba, ei):
    return pl.pallas_call(
        _tc_body,
        in_specs=[
            pl.BlockSpec((N, D), lambda: (0, 0)),
            pl.BlockSpec((D, D), lambda: (0, 0)),
            pl.BlockSpec((1, D), lambda: (0, 0)),
            pl.BlockSpec((8, D), lambda: (0, 0)),
            pl.BlockSpec(memory_space=pltpu.SMEM),
            pl.BlockSpec((2, E), lambda: (0, 0)),
        ],
        out_specs=[
            pl.BlockSpec((N, D), lambda: (0, 0)),
            pl.BlockSpec((8, NP), lambda: (0, 0)),
            pl.BlockSpec((1, E), lambda: (0, 0)),
        ],
        out_shape=[
            jax.ShapeDtypeStruct((N, D), jnp.float32),
            jax.ShapeDtypeStruct((8, NP), jnp.float32),
            jax.ShapeDtypeStruct((1, E), jnp.int32),
        ],
    )(obs, wfc, bfc, w12, ba, ei)


SCAN_UNROLL = 4


@functools.partial(
    pl.kernel,
    out_type=jax.ShapeDtypeStruct((N, D), jnp.float32),
    mesh=plsc.VectorSubcoreMesh(core_axis_name="c", subcore_axis_name="s"),
    compiler_params=pltpu.CompilerParams(needs_layout_passes=False),
    scratch_types=[
        pltpu.VMEM((ROWS_T + 1, D), jnp.float32),
        pltpu.VMEM((NP,), jnp.float32),
        pltpu.VMEM((QPAD,), jnp.float32),
        pltpu.VMEM((2 * W,), jnp.int32),
        pltpu.VMEM((CAP,), jnp.int32),
        pltpu.VMEM((PIPE * EC, D), jnp.float32),
        pltpu.VMEM((PIPE * EC,), jnp.float32),
        pltpu.VMEM((PIPE * EC,), jnp.int32),
        pltpu.VMEM((PIPE * EC,), jnp.int32),
        pltpu.SemaphoreType.DMA((2,)),
        pltpu.SemaphoreType.DMA((PIPE,)),
    ],
)
def _sc_gat(feat_hbm, pq_hbm, qt_hbm, pk_hbm, out_hbm,
            acc, p_v, q_v, pk_w, esel, rows_v, alpha_v, dloc_v,
            gidx_v, wsem, gsem):
    c = lax.axis_index("c")
    s = lax.axis_index("s")
    wid = c * NS + s
    lo = wid * ROWS_T
    hi = lo + ROWS_T
    iota = lax.broadcasted_iota(jnp.int32, (LANES,), 0)
    zerosf = jnp.zeros((LANES,), jnp.float32)
    zeros16 = jnp.zeros((LANES,), jnp.int32)
    cols = [jj * LANES + iota for jj in range(D // LANES)]

    def zrow(r, carry):
        for j in range(D // LANES):
            acc[r, pl.ds(j * LANES, LANES)] = zerosf
        return carry
    lax.fori_loop(0, ROWS_T + 1, zrow, 0)

    pltpu.sync_copy(pq_hbm.at[0], p_v)
    pltpu.sync_copy(qt_hbm.at[wid], q_v)

    def start_gather(par):
        pltpu.async_copy(feat_hbm.at[gidx_v.at[pl.ds(par * EC, EC)]],
                         rows_v.at[pl.ds(par * EC, EC)], gsem.at[par])

    def accumulate(par):
        pltpu.make_async_copy(feat_hbm.at[pl.ds(0, EC)],
                              rows_v.at[pl.ds(par * EC, EC)],
                              gsem.at[par]).wait()

        def srow(e2, carry):
            for u in range(2):
                pe = par * EC + e2 * 2 + u
                ab = plsc.load_gather(alpha_v, [zeros16 + pe])
                dlb = plsc.load_gather(dloc_v, [zeros16 + pe])
                vs = [rows_v[pe, pl.ds(jj * LANES, LANES)] * ab
                      for jj in range(D // LANES)]
                for jj in range(D // LANES):
                    plsc.addupdate_scatter(acc, [dlb, cols[jj]], vs[jj])
            return carry
        lax.fori_loop(0, EC // 2, srow, 0)

    def prep_groups(par, base, npend):
        for g in range(EC // LANES):
            pk16 = esel[pl.ds(base + g * LANES, LANES)]
            s16 = lax.shift_right_logical(pk16, 14)
            dl16 = (pk16 & (PSH - 1)) - lo
            if npend is not None:
                valid = (g * LANES + iota) < npend
                s16 = jnp.where(valid, s16, 0)
                dl16 = jnp.where(valid, dl16, TRASH)
            pg = plsc.load_gather(p_v, [s16])
            qg = plsc.load_gather(q_v, [dl16])
            a = 1.0 / (1.0 + jnp.exp(-(pg + qg)))
            if npend is not None:
                a = jnp.where(valid, a, 0.0)
            alpha_v[pl.ds(par * EC + g * LANES, LANES)] = a
            dloc_v[pl.ds(par * EC + g * LANES, LANES)] = dl16
            gidx_v[pl.ds(par * EC + g * LANES, LANES)] = s16

    def prep(par, base):
        prep_groups(par, base, None)

    def prep_masked(par, npend):
        prep_groups(par, jnp.int32(0), npend)

    pltpu.async_copy(pk_hbm.at[pl.ds(0, W)], pk_w.at[pl.ds(0, W)],
                     wsem.at[0])

    def window(w, st):
        pend0, nq0, head0 = st
        cur = w % 2
        woff = w * W
        pltpu.make_async_copy(pk_hbm.at[pl.ds(woff, W)],
                              pk_w.at[pl.ds(cur * W, W)],
                              wsem.at[cur]).wait()

        @pl.when(w + 1 < NW)
        def _prefetch():
            noff = (w + 1) * W
            pltpu.async_copy(pk_hbm.at[pl.ds(noff, W)],
                             pk_w.at[pl.ds((1 - cur) * W, W)],
                             wsem.at[1 - cur])

        def scan(g, pendc):
            pks, ms, offs = [], [], []
            for u in range(SCAN_UNROLL):
                pk16 = pk_w[pl.ds(cur * W + (g * SCAN_UNROLL + u) * LANES,
                                  LANES)]
                d16 = pk16 & (PSH - 1)
                m = (d16 >= lo) & (d16 < hi)
                pks.append(pk16)
                ms.append(m)
                offs.append(pendc)
                pendc = pendc + plsc.all_reduce_population_count(m)[0]
            for u in range(SCAN_UNROLL):
                plsc.store_compressed(esel.at[pl.ds(offs[u], LANES)],
                                      pks[u], mask=ms[u])
            return pendc
        pend = lax.fori_loop(0, W // (LANES * SCAN_UNROLL), scan, pend0)

        def pcond(pst):
            return pst[0] >= EC

        def pbody(pst):
            pendc, basec, nqc, headc = pst

            @pl.when(nqc == PIPE)
            def _drain():
                accumulate(headc)
            nq2 = jnp.where(nqc == PIPE, nqc - 1, nqc)
            prep(headc, basec)
            start_gather(headc)
            return (pendc - EC, basec + EC, nq2 + 1,
                    (headc + 1) & (PIPE - 1))
        pend, base, nq, head = lax.while_loop(
            pcond, pbody, (pend, jnp.int32(0), nq0, head0))

        for k in range(EC // LANES):
            idx16 = base + k * LANES + iota
            ev = plsc.load_gather(esel, [idx16])
            esel[pl.ds(k * LANES, LANES)] = ev
        return (pend, nq, head)

    pend, nq, head = lax.fori_loop(
        0, NW, window, (jnp.int32(0), jnp.int32(0), jnp.int32(0)))

    @pl.when(pend > 0)
    def _tail():
        @pl.when(nq == PIPE)
        def _oldest():
            accumulate(head)
        prep_masked(head, pend)
        start_gather(head)

    nq = jnp.where(pend > 0, jnp.where(nq == PIPE, nq, nq + 1), nq)
    head = jnp.where(pend > 0, (head + 1) & (PIPE - 1), head)

    def dbody(k, carry):
        accumulate((head + PIPE - nq + k) & (PIPE - 1))
        return carry
    lax.fori_loop(0, nq, dbody, 0)

    @pl.when(wid < NT - 1)
    def _full():
        pltpu.sync_copy(acc.at[pl.ds(0, ROWS_T)],
                        out_hbm.at[pl.ds(lo, ROWS_T)])

    @pl.when(wid == NT - 1)
    def _last():
        pltpu.sync_copy(acc.at[pl.ds(0, LAST_ROWS)],
                        out_hbm.at[pl.ds(lo, LAST_ROWS)])


def kernel(observations, edge_index, W_fc, b_fc, W_attn, b_attn):
    w1 = W_attn[0, :D]
    w2 = W_attn[0, D:]
    w12 = jnp.zeros((8, D), jnp.float32).at[0].set(w1).at[1].set(w2)
    feat, pq, pk = _tc_features(
        observations, W_fc, b_fc.reshape(1, D), w12, b_attn.reshape(1, 1),
        edge_index)
    qt = jnp.pad(pq[1].reshape(NT, ROWS_T),
                 ((0, 0), (0, QPAD - ROWS_T)))
    return _sc_gat(feat, pq, qt, pk.reshape(E))
